# baseline (speedup 1.0000x reference)
import functools

import jax
import jax.numpy as jnp
from jax import lax
from jax.experimental import pallas as pl
from jax.experimental.pallas import tpu as pltpu
from jax.experimental.pallas import tpu_sc as plsc

NC = 2
NS = 16
NW = NC * NS
CW = 3584


def _mesh():
    return plsc.VectorSubcoreMesh(
        core_axis_name="c", subcore_axis_name="s",
        num_cores=NC, num_subcores=NS)


@jax.jit
def _linearize(table_t, tail_p):
    n_coords, n_boxes = table_t.shape
    tcols = -(-n_boxes // 128)
    tc_per_w = -(-tcols // NW)
    slab = tc_per_w * 128
    n_chunks = -(-slab // CW)
    aligned = (n_boxes // 128) * 128
    stride = aligned + (128 if n_boxes > aligned else 0)

    @functools.partial(
        pl.kernel,
        out_type=jax.ShapeDtypeStruct((n_coords * stride,), jnp.float32),
        mesh=_mesh(),
        scratch_types=[
            pltpu.VMEM((n_coords, CW), jnp.float32),
            pltpu.VMEM((n_coords, 128), jnp.float32),
            pltpu.SemaphoreType.DMA,
        ],
    )
    def k(table_hbm, tail_hbm, out_hbm, buf_v, tbuf_v, sem0):
        w = lax.axis_index("s") * NC + lax.axis_index("c")
        base = w * slab
        hi = aligned - CW

        def body(j, _):
            off = jnp.minimum(base + j * CW, hi)
            off = pl.multiple_of(off, 128)
            pltpu.sync_copy(table_hbm.at[:, pl.ds(off, CW)], buf_v)
            copies = [
                pltpu.async_copy(
                    buf_v.at[r], out_hbm.at[pl.ds(r * stride + off, CW)],
                    sem0)
                for r in range(n_coords)
            ]
            for c in copies:
                c.wait()
            return 0
        lax.fori_loop(0, n_chunks, body, 0)
        if stride > aligned:
            @pl.when(w == NW - 1)
            def _():
                pltpu.sync_copy(tail_hbm, tbuf_v)
                copies = [
                    pltpu.async_copy(
                        tbuf_v.at[r],
                        out_hbm.at[pl.ds(r * stride + aligned, 128)], sem0)
                    for r in range(n_coords)
                ]
                for c in copies:
                    c.wait()

    return k(table_t, tail_p)


@jax.jit
def _gather(ids, table_lin4):
    batch = ids.shape[0]
    n_coords = NW
    stride4 = table_lin4.shape[0] * 4 // n_coords
    L = 16

    @functools.partial(
        pl.kernel,
        out_type=jax.ShapeDtypeStruct((n_coords * batch,), jnp.float32),
        mesh=_mesh(),
        scratch_types=[
            pltpu.VMEM((batch,), jnp.int32),
            pltpu.VMEM((batch // 2,), jnp.int32),
            pltpu.VMEM((batch // 2, 4), jnp.float32),
            pltpu.VMEM((batch,), jnp.float32),
            pltpu.SemaphoreType.DMA,
        ],
        compiler_params=pltpu.CompilerParams(
            use_tc_tiling_on_sc=False, needs_layout_passes=False),
    )
    def k(ids_hbm, table_hbm, out_hbm, ids_v, idx_v, g_v, o_v, sem):
        w = lax.axis_index("s") * NC + lax.axis_index("c")
        pltpu.sync_copy(ids_hbm, ids_v)
        wbase = w * stride4
        half = batch // 2
        iota = lax.iota(jnp.int32, L)
        for h in range(2):
            hb = h * half
            def mkidx(t, _):
                v = ids_v[pl.ds(hb + t * L, L)]
                idx_v[pl.ds(t * L, L)] = lax.shift_right_logical(v + wbase, 2)
                return 0
            lax.fori_loop(0, half // L, mkidx, 0)
            pltpu.async_copy(table_hbm.at[idx_v], g_v, sem).wait()
            def sel(t, _):
                kv = t * L + iota
                v = ids_v[pl.ds(hb + t * L, L)]
                x = plsc.load_gather(g_v, [kv, lax.bitwise_and(v, 3)])
                plsc.store_scatter(o_v, [hb + kv], x)
                return 0
            lax.fori_loop(0, half // L, sel, 0)
        pltpu.sync_copy(o_v, out_hbm.at[pl.ds(w * batch, batch)])

    return k(ids, table_lin4)


def kernel(ids, boxes):
    num_models, num_boxes, two, dim = boxes.shape
    batch = ids.shape[0]
    n_coords = num_models * two * dim
    table_t = jnp.transpose(boxes, (0, 2, 3, 1)).reshape(n_coords, num_boxes)
    aligned = (num_boxes // 128) * 128
    stride = aligned + (128 if num_boxes > aligned else 0)
    tail_p = jnp.pad(table_t[:, aligned:], ((0, 0), (0, stride - num_boxes)))
    table_lin4 = _linearize(table_t, tail_p).reshape(n_coords * stride // 4, 4)
    out_flat = _gather(ids, table_lin4)
    return out_flat.reshape(num_models, two, dim, batch).transpose(0, 3, 1, 2)


# speedup vs baseline: 64.8094x; 64.8094x over previous
import functools

import jax
import jax.numpy as jnp
from jax import lax
from jax.experimental import pallas as pl
from jax.experimental.pallas import tpu as pltpu
from jax.experimental.pallas import tpu_sc as plsc

NC = 2
NS = 16
NW = NC * NS
CW = 3840


def _mesh():
    return plsc.VectorSubcoreMesh(
        core_axis_name="c", subcore_axis_name="s",
        num_cores=NC, num_subcores=NS)


@jax.jit
def _linearize(table_t, tail_p):
    n_coords, n_boxes = table_t.shape
    tcols = -(-n_boxes // 128)
    tc_per_w = -(-tcols // NW)
    slab = tc_per_w * 128
    n_chunks = -(-slab // CW)
    aligned = (n_boxes // 128) * 128
    stride = aligned + (128 if n_boxes > aligned else 0)

    @functools.partial(
        pl.kernel,
        out_type=jax.ShapeDtypeStruct((n_coords * stride,), jnp.float32),
        mesh=_mesh(),
        scratch_types=[
            pltpu.VMEM((n_coords, CW), jnp.float32),
            pltpu.VMEM((n_coords, 128), jnp.float32),
            pltpu.SemaphoreType.DMA,
        ],
    )
    def k(table_hbm, tail_hbm, out_hbm, buf_v, tbuf_v, sem0):
        w = lax.axis_index("s") * NC + lax.axis_index("c")
        base = w * slab
        hi = aligned - CW

        def body(j, _):
            off = jnp.minimum(base + j * CW, hi)
            off = pl.multiple_of(off, 128)
            pltpu.sync_copy(table_hbm.at[:, pl.ds(off, CW)], buf_v)
            copies = [
                pltpu.async_copy(
                    buf_v.at[r], out_hbm.at[pl.ds(r * stride + off, CW)],
                    sem0)
                for r in range(n_coords)
            ]
            for c in copies:
                c.wait()
            return 0
        lax.fori_loop(0, n_chunks, body, 0)
        if stride > aligned:
            @pl.when(w == NW - 1)
            def _():
                pltpu.sync_copy(tail_hbm, tbuf_v)
                copies = [
                    pltpu.async_copy(
                        tbuf_v.at[r],
                        out_hbm.at[pl.ds(r * stride + aligned, 128)], sem0)
                    for r in range(n_coords)
                ]
                for c in copies:
                    c.wait()

    return k(table_t, tail_p)


@jax.jit
def _gather(ids, table_lin2d):
    n_coords = table_lin2d.shape[0]
    batch = ids.shape[0]

    @functools.partial(
        pl.kernel,
        out_type=jax.ShapeDtypeStruct((n_coords * batch,), jnp.float32),
        mesh=_mesh(),
        scratch_types=[
            pltpu.VMEM((batch,), jnp.int32),
            pltpu.VMEM((batch,), jnp.float32),
            pltpu.SemaphoreType.DMA,
        ],
        compiler_params=pltpu.CompilerParams(use_tc_tiling_on_sc=False),
    )
    def k(ids_hbm, table_hbm, out_hbm, idx_v, o_v, sem):
        w = lax.axis_index("s") * NC + lax.axis_index("c")
        pltpu.sync_copy(ids_hbm, idx_v)
        pltpu.async_copy(table_hbm.at[w].at[idx_v], o_v, sem).wait()
        pltpu.sync_copy(o_v, out_hbm.at[pl.ds(w * batch, batch)])

    return k(ids, table_lin2d)


def kernel(ids, boxes):
    num_models, num_boxes, two, dim = boxes.shape
    batch = ids.shape[0]
    n_coords = num_models * two * dim
    table_t = jnp.transpose(boxes, (0, 2, 3, 1)).reshape(n_coords, num_boxes)
    aligned = (num_boxes // 128) * 128
    stride = aligned + (128 if num_boxes > aligned else 0)
    tail_p = jnp.pad(table_t[:, aligned:], ((0, 0), (0, stride - num_boxes)))
    table_lin = _linearize(table_t, tail_p).reshape(n_coords, stride)
    out_flat = _gather(ids, table_lin)
    return out_flat.reshape(num_models, two, dim, batch).transpose(0, 3, 1, 2)


# speedup vs baseline: 67.5228x; 1.0419x over previous
import functools

import jax
import jax.numpy as jnp
from jax import lax
from jax.experimental import pallas as pl
from jax.experimental.pallas import tpu as pltpu
from jax.experimental.pallas import tpu_sc as plsc

NC = 2
NS = 16
NW = NC * NS
CW = 3584


def _mesh():
    return plsc.VectorSubcoreMesh(
        core_axis_name="c", subcore_axis_name="s",
        num_cores=NC, num_subcores=NS)


@jax.jit
def _linearize(table_t, tail_p):
    n_coords, n_boxes = table_t.shape
    tcols = -(-n_boxes // 128)
    tc_per_w = -(-tcols // NW)
    slab = tc_per_w * 128
    n_chunks = -(-slab // CW)
    aligned = (n_boxes // 128) * 128
    stride = aligned + (128 if n_boxes > aligned else 0)

    @functools.partial(
        pl.kernel,
        out_type=jax.ShapeDtypeStruct((n_coords * stride,), jnp.float32),
        mesh=_mesh(),
        scratch_types=[
            pltpu.VMEM((n_coords, CW), jnp.float32),
            pltpu.VMEM((n_coords, 128), jnp.float32),
            pltpu.SemaphoreType.DMA,
        ],
    )
    def k(table_hbm, tail_hbm, out_hbm, buf_v, tbuf_v, sem0):
        w = lax.axis_index("s") * NC + lax.axis_index("c")
        base = w * slab
        hi = aligned - CW

        def body(j, _):
            off = jnp.minimum(base + j * CW, hi)
            off = pl.multiple_of(off, 128)
            pltpu.sync_copy(table_hbm.at[:, pl.ds(off, CW)], buf_v)
            copies = [
                pltpu.async_copy(
                    buf_v.at[r], out_hbm.at[pl.ds(r * stride + off, CW)],
                    sem0)
                for r in range(n_coords)
            ]
            for c in copies:
                c.wait()
            return 0
        lax.fori_loop(0, n_chunks, body, 0)
        if stride > aligned:
            @pl.when(w == NW - 1)
            def _():
                pltpu.sync_copy(tail_hbm, tbuf_v)
                copies = [
                    pltpu.async_copy(
                        tbuf_v.at[r],
                        out_hbm.at[pl.ds(r * stride + aligned, 128)], sem0)
                    for r in range(n_coords)
                ]
                for c in copies:
                    c.wait()

    return k(table_t, tail_p)


@jax.jit
def _gather(ids, table_lin2d):
    n_coords = table_lin2d.shape[0]
    batch = ids.shape[0]

    @functools.partial(
        pl.kernel,
        out_type=jax.ShapeDtypeStruct((n_coords * batch,), jnp.float32),
        mesh=_mesh(),
        scratch_types=[
            pltpu.VMEM((batch,), jnp.int32),
            pltpu.VMEM((batch,), jnp.float32),
            pltpu.SemaphoreType.DMA,
        ],
        compiler_params=pltpu.CompilerParams(use_tc_tiling_on_sc=False),
    )
    def k(ids_hbm, table_hbm, out_hbm, idx_v, o_v, sem):
        w = lax.axis_index("s") * NC + lax.axis_index("c")
        pltpu.sync_copy(ids_hbm, idx_v)
        pltpu.async_copy(table_hbm.at[w].at[idx_v], o_v, sem).wait()
        pltpu.sync_copy(o_v, out_hbm.at[pl.ds(w * batch, batch)])

    return k(ids, table_lin2d)


def kernel(ids, boxes):
    num_models, num_boxes, two, dim = boxes.shape
    batch = ids.shape[0]
    n_coords = num_models * two * dim
    table_t = jnp.transpose(boxes, (0, 2, 3, 1)).reshape(n_coords, num_boxes)
    aligned = (num_boxes // 128) * 128
    stride = aligned + (128 if num_boxes > aligned else 0)
    tail_p = jnp.pad(table_t[:, aligned:], ((0, 0), (0, stride - num_boxes)))
    table_lin = _linearize(table_t, tail_p).reshape(n_coords, stride)
    out_flat = _gather(ids, table_lin)
    return out_flat.reshape(num_models, two, dim, batch).transpose(0, 3, 1, 2)
